# baseline (speedup 1.0000x reference)
import functools

import jax
import jax.numpy as jnp
from jax import lax
from jax.experimental import pallas as pl
from jax.experimental.pallas import tpu as pltpu, tpu_sc as plsc

BLK = 128
NB = 16
KEEP = 2
HPP = 4
NEG = -1e9
FMIN = -3.0e38


def _probs_kernel(q_ref, k_ref, p_ref):
    q = q_ref[0, 0]
    k = k_ref[0, 0]
    S, D = q.shape
    scale = jnp.float32(1.0) / jnp.sqrt(jnp.float32(D))
    qp = jnp.concatenate(
        [jnp.sum(q[i * BLK:(i + 1) * BLK, :], axis=0, keepdims=True)
         for i in range(NB)], axis=0) * jnp.float32(1.0 / BLK)
    kp = jnp.concatenate(
        [jnp.sum(k[i * BLK:(i + 1) * BLK, :], axis=0, keepdims=True)
         for i in range(NB)], axis=0) * jnp.float32(1.0 / BLK)
    s = jax.lax.dot_general(qp.astype(jnp.bfloat16), kp.astype(jnp.bfloat16),
                            (((1,), (1,)), ((), ())),
                            preferred_element_type=jnp.float32) * scale
    m = jnp.max(s, axis=1, keepdims=True)
    e = jnp.exp(s - m)
    p_ref[0] = e / jnp.sum(e, axis=1, keepdims=True)


def _make_sc_top2(nh):
    info = plsc.get_sparse_core_info()
    nc = info.num_cores
    mesh = plsc.VectorSubcoreMesh(core_axis_name="c", subcore_axis_name="s")

    @functools.partial(
        pl.kernel, mesh=mesh,
        out_type=jax.ShapeDtypeStruct((nh, 2, NB), jnp.int32),
        scratch_types=[
            pltpu.VMEM((NB, NB), jnp.float32),
            pltpu.VMEM((2, NB), jnp.int32),
            pltpu.SemaphoreType.DMA,
        ],
    )
    def sc_top2(p_hbm, out_hbm, p_v, idx_v, sem):
        wid = lax.axis_index("s") * nc + lax.axis_index("c")

        @pl.when(wid < nh)
        def _():
            pltpu.sync_copy(p_hbm.at[wid], p_v)
            iota = lax.iota(jnp.int32, NB)
            rots = [jnp.remainder(iota + sh, NB) for sh in (8, 4, 2, 1)]

            def allmax(x):
                for rot in rots:
                    x = jnp.maximum(x, x[rot])
                return x

            def allmin(x):
                for rot in rots:
                    x = jnp.minimum(x, x[rot])
                return x

            def argmax_splat(x):
                m = allmax(x)
                return allmin(jnp.where(x >= m, iota, NB))

            a1v = jnp.zeros((NB,), jnp.int32)
            a2v = jnp.zeros((NB,), jnp.int32)
            for r in range(NB):
                row = p_v[r, :]
                a1 = argmax_splat(row)
                row2 = jnp.where(iota == a1, jnp.float32(FMIN), row)
                a2 = argmax_splat(row2)
                a1v = jnp.where(iota == r, a1, a1v)
                a2v = jnp.where(iota == r, a2, a2v)
            idx_v[0, :] = a1v
            idx_v[1, :] = a2v
            pltpu.sync_copy(idx_v, out_hbm.at[wid])

    return sc_top2


def _attn_kernel(idx_ref, q_ref, k_ref, v_ref, o_ref):
    b = pl.program_id(0)
    g = pl.program_id(1)
    scale_a = jnp.float32(0.125)
    for hh in range(HPP):
        row = (b * pl.num_programs(1) + g) * HPP + hh
        for qb in range(NB):
            i0 = idx_ref[row, 0, qb]
            i1 = idx_ref[row, 1, qb]
            qblk = q_ref[0, hh, qb * BLK:(qb + 1) * BLK, :]
            kc = jnp.concatenate(
                [k_ref[0, hh, pl.ds(i0 * BLK, BLK), :],
                 k_ref[0, hh, pl.ds(i1 * BLK, BLK), :],
                 k_ref[0, hh, qb * BLK:(qb + 1) * BLK, :]], axis=0)
            vc = jnp.concatenate(
                [v_ref[0, hh, pl.ds(i0 * BLK, BLK), :],
                 v_ref[0, hh, pl.ds(i1 * BLK, BLK), :],
                 v_ref[0, hh, qb * BLK:(qb + 1) * BLK, :]], axis=0)
            sc = jnp.dot(qblk, kc.T,
                         preferred_element_type=jnp.float32) * scale_a
            dup = jnp.logical_or(i0 == qb, i1 == qb)
            colmask = (jax.lax.broadcasted_iota(jnp.int32, (1, 3 * BLK), 1)
                       >= 2 * BLK)
            sc = jnp.where(jnp.logical_and(dup, colmask), NEG, sc)
            mx = jnp.max(sc, axis=1, keepdims=True)
            pr = jnp.exp(sc - mx)
            denom = jnp.sum(pr, axis=1, keepdims=True)
            acc = jnp.dot(pr, vc, preferred_element_type=jnp.float32)
            o_ref[0, hh, qb * BLK:(qb + 1) * BLK, :] = acc / denom


def kernel(q, k, v):
    B, H, S, D = q.shape

    p = pl.pallas_call(
        _probs_kernel,
        grid=(B, H),
        in_specs=[
            pl.BlockSpec((1, 1, S, D), lambda b, h: (b, h, 0, 0)),
            pl.BlockSpec((1, 1, S, D), lambda b, h: (b, h, 0, 0)),
        ],
        out_specs=pl.BlockSpec((1, NB, NB), lambda b, h: (b * H + h, 0, 0)),
        out_shape=jax.ShapeDtypeStruct((B * H, NB, NB), jnp.float32),
    )(q, k)

    idx = _make_sc_top2(B * H)(p)

    out = pl.pallas_call(
        _attn_kernel,
        grid_spec=pltpu.PrefetchScalarGridSpec(
            num_scalar_prefetch=1,
            grid=(B, H // HPP),
            in_specs=[
                pl.BlockSpec((1, HPP, S, D), lambda b, g, i: (b, g, 0, 0)),
                pl.BlockSpec((1, HPP, S, D), lambda b, g, i: (b, g, 0, 0)),
                pl.BlockSpec((1, HPP, S, D), lambda b, g, i: (b, g, 0, 0)),
            ],
            out_specs=pl.BlockSpec((1, HPP, S, D), lambda b, g, i: (b, g, 0, 0)),
        ),
        out_shape=jax.ShapeDtypeStruct((B, H, S, D), jnp.float32),
    )(idx, q, k, v)

    return out
